# baseline (speedup 1.0000x reference)
import functools

import jax
import jax.numpy as jnp
from jax import lax
from jax.experimental import pallas as pl
from jax.experimental.pallas import tpu as pltpu
from jax.experimental.pallas import tpu_sc as plsc

B, L = 4096, 200
V, H = 1000000, 64
TOTAL = B * L
N_AGE, N_POS, N_PT = 100, 200, 3
N_AGE_PAD = 104
N_TAB = N_PT * N_POS * N_AGE


def _t2v_body(wa_ref, ba_ref, wp_ref, bp_ref, aa_ref, ab_ref,
              ta_ref, tp_ref):
    aw = aa_ref[0, 0]
    a = lax.broadcasted_iota(jnp.int32, (N_AGE_PAD, H), 0).astype(jnp.float32)
    cola = lax.broadcasted_iota(jnp.int32, (N_AGE_PAD, H), 1)
    arg_a = a * wa_ref[...] + ba_ref[...]
    ta_ref[...] = aw * jnp.where(cola == H - 1, arg_a, jnp.cos(arg_a))
    pw = ab_ref[0, 0]
    p = lax.broadcasted_iota(jnp.int32, (N_POS, H), 0).astype(jnp.float32)
    colp = lax.broadcasted_iota(jnp.int32, (N_POS, H), 1)
    arg_p = p * wp_ref[...] + bp_ref[...]
    tp_ref[...] = pw * jnp.where(colp == H - 1, arg_p, jnp.sin(arg_p))


def _build_t2v(wa, ba, wp, bp, aa, ab):
    w_spec = pl.BlockSpec((1, H), lambda: (0, 0))
    s_spec = pl.BlockSpec((1, 1), lambda: (0, 0), memory_space=pltpu.SMEM)
    return pl.pallas_call(
        _t2v_body,
        in_specs=[w_spec, w_spec, w_spec, w_spec, s_spec, s_spec],
        out_specs=[pl.BlockSpec((N_AGE_PAD, H), lambda: (0, 0)),
                   pl.BlockSpec((N_POS, H), lambda: (0, 0))],
        out_shape=[jax.ShapeDtypeStruct((N_AGE_PAD, H), jnp.float32),
                   jax.ShapeDtypeStruct((N_POS, H), jnp.float32)],
    )(wa, ba, wp, bp, aa, ab)


RT = N_POS * N_AGE
IDX_R, IDX_C = 6400, 128
IDX_GRID = 50
IDX_BLK = IDX_R // IDX_GRID


def _fuse_body(tp_ref, ptab_ref, tpa_ref):
    p = pl.program_id(0)
    pt_row = jnp.where(p == 0, ptab_ref[0:1, :],
                       jnp.where(p == 1, ptab_ref[1:2, :], ptab_ref[2:3, :]))
    tpa_ref[...] = tp_ref[...] + pt_row


def _build_dense(tp, ptab):
    return pl.pallas_call(
        _fuse_body,
        grid=(N_PT,),
        in_specs=[pl.BlockSpec((N_POS, H), lambda i: (0, 0)),
                  pl.BlockSpec((8, H), lambda i: (0, 0))],
        out_specs=pl.BlockSpec((N_POS, H), lambda i: (i, 0)),
        out_shape=jax.ShapeDtypeStruct((N_PT * N_POS, H), jnp.float32),
    )(tp, ptab)


def _fuseidx_body(pos_ref, pt_ref, fused_ref):
    fused_ref[...] = pt_ref[...] * N_POS + pos_ref[...]


def _build_fused_idx(pos2, pt2):
    idx_spec = pl.BlockSpec((IDX_BLK, IDX_C), lambda i: (i, 0))
    return pl.pallas_call(
        _fuseidx_body,
        grid=(IDX_GRID,),
        in_specs=[idx_spec, idx_spec],
        out_specs=idx_spec,
        out_shape=jax.ShapeDtypeStruct((IDX_R, IDX_C), jnp.int32),
    )(pos2, pt2)


NC, NS = 2, 16
NW = NC * NS
HG, BG = H // 8, B // 128
UNITS = L * BG
U_PER_W = UNITS // NW
CU = 2
CBB = CU * 128
NCHUNK = U_PER_W // CU

_sc_mesh = plsc.VectorSubcoreMesh(core_axis_name="c", subcore_axis_name="s")


@functools.partial(
    pl.kernel,
    out_type=jax.ShapeDtypeStruct((L, HG, BG, 8, 128), jnp.float32),
    mesh=_sc_mesh,
    scratch_types=[
        pltpu.VMEM((2, CBB), jnp.int32),
        pltpu.VMEM((2, CBB), jnp.int32),
        pltpu.VMEM((2, CBB), jnp.int32),
        pltpu.VMEM((2, CBB, H), jnp.float32),
        pltpu.VMEM((2, HG, CU, 8, 128), jnp.float32),
        pltpu.VMEM((N_PT * N_POS, H), jnp.float32),
        pltpu.VMEM((N_AGE_PAD, H), jnp.float32),
    ] + [pltpu.SemaphoreType.DMA] * 7,
    compiler_params=pltpu.CompilerParams(use_tc_tiling_on_sc=False,
                                         needs_layout_passes=False),
)
def _sc_gather(tok_hbm, pa_hbm, ag_hbm, table_hbm, tpa_hbm, a1_hbm, out_hbm,
               tok_v, pa_v, ag_v, rows_v, t_v, tpa_t, a1_t, *sems):
    wid = lax.axis_index("s") * NC + lax.axis_index("c")
    semI = sems[0:2]
    semA = sems[2:4]
    semC = sems[4:6]
    semT = sems[6]

    def u0_of(ci):
        return wid * U_PER_W + ci * CU

    def idx_copies(ci, s):
        e0 = pl.multiple_of(u0_of(ci) * 128, 8)
        return (pltpu.make_async_copy(tok_hbm.at[pl.ds(e0, CBB)], tok_v.at[s],
                                      semI[s]),
                pltpu.make_async_copy(pa_hbm.at[pl.ds(e0, CBB)], pa_v.at[s],
                                      semI[s]),
                pltpu.make_async_copy(ag_hbm.at[pl.ds(e0, CBB)], ag_v.at[s],
                                      semI[s]))

    def a_copies(s):
        return [pltpu.make_async_copy(
            table_hbm.at[tok_v.at[s]], rows_v.at[s], semA[s])]

    def c_copy(ci, s):
        u0 = u0_of(ci)
        l = u0 // BG
        bg = u0 % BG
        return pltpu.make_async_copy(t_v.at[s],
                                     out_hbm.at[l, :, pl.ds(bg, CU)],
                                     semC[s])

    lane = lax.iota(jnp.int32, 16)

    def transpose_add(s):
        buf = rows_v.at[s]
        tb = t_v.at[s]
        pas = pa_v.at[s]
        ags = ag_v.at[s]

        def mbody(m, carry):
            h0 = m * 16

            def kbody(k, c2):
                for u in range(2):
                    kk = 2 * k + u
                    bv = lane + kk * 16
                    bv7 = bv >> 7
                    bv127 = bv & 127
                    pa16 = pas[pl.ds(kk * 16, 16)]
                    ag16 = ags[pl.ds(kk * 16, 16)]
                    for d in range(16):
                        hv = h0 + ((lane + d) & 15)
                        v = (plsc.load_gather(buf, [bv, hv])
                             + plsc.load_gather(tpa_t, [pa16, hv])
                             + plsc.load_gather(a1_t, [ag16, hv]))
                        plsc.store_scatter(tb, [hv >> 3, bv7, hv & 7, bv127],
                                           v)
                return c2

            return lax.fori_loop(0, CBB // 32, kbody, carry)

        lax.fori_loop(0, H // 16, mbody, 0)

    def fire(cps, add=False):
        for cp in (cps if isinstance(cps, (list, tuple)) else [cps]):
            cp.start(add=add)

    def wait(cps):
        for cp in (cps if isinstance(cps, (list, tuple)) else [cps]):
            cp.wait()

    tl1 = pltpu.make_async_copy(tpa_hbm, tpa_t, semT)
    tl2 = pltpu.make_async_copy(a1_hbm, a1_t, semT)
    fire([tl1, tl2])
    wait([tl1, tl2])

    def sub(ci, s, cwait=True, prefetch=True):
        wait(idx_copies(ci, s))
        fire(a_copies(s))
        wait(a_copies(1 - s))
        if cwait:
            wait(c_copy(ci - 3, 1 - s))
        transpose_add(1 - s)
        fire(c_copy(ci - 1, 1 - s))
        if prefetch:
            fire(idx_copies(ci + 1, 1 - s))

    fire(idx_copies(0, 0))
    wait(idx_copies(0, 0))
    fire(a_copies(0))
    fire(idx_copies(1, 1))
    sub(1, 1, cwait=False)
    sub(2, 0, cwait=False)

    def body(t, carry):
        sub(2 * t + 3, 1)
        sub(2 * t + 4, 0)
        return carry

    lax.fori_loop(0, (NCHUNK - 4) // 2, body, 0)

    last = NCHUNK - 1
    sub(last, 1, prefetch=False)
    wait(a_copies(1))
    wait(c_copy(last - 2, 1))
    transpose_add(1)
    fire(c_copy(last, 1))
    wait(c_copy(last - 1, 0))
    wait(c_copy(last, 1))


def kernel(tokens, position, age, partner_type, token_table, partner_table,
           age_w, age_b, age_w0, age_b0, abs_w, abs_b, abs_w0, abs_b0,
           res_age_alpha, res_abs_alpha):
    f32 = jnp.float32
    ag2 = age.astype(jnp.int32).T.reshape(-1)
    pos2 = position.astype(jnp.int32).T.reshape(IDX_R, IDX_C)
    pt2 = partner_type.astype(jnp.int32).T.reshape(IDX_R, IDX_C)
    tok2 = tokens.astype(jnp.int32).T.reshape(-1)
    wa = jnp.concatenate([age_w, age_w0], axis=1).astype(f32)
    ba = jnp.concatenate([age_b, age_b0], axis=1).astype(f32)
    wp = jnp.concatenate([abs_w, abs_w0], axis=1).astype(f32)
    bp = jnp.concatenate([abs_b, abs_b0], axis=1).astype(f32)
    ptab = jnp.pad(partner_table.astype(f32), ((0, 8 - 3), (0, 0)))
    aa = res_age_alpha.astype(f32).reshape(1, 1)
    ab = res_abs_alpha.astype(f32).reshape(1, 1)
    ta, tp = _build_t2v(wa, ba, wp, bp, aa, ab)
    tpa = _build_dense(tp, ptab)
    pa = _build_fused_idx(pos2, pt2).reshape(-1)
    out5 = _sc_gather(tok2, pa, ag2, token_table.astype(f32), tpa, ta)
    return out5.transpose(2, 4, 0, 1, 3).reshape(B, L, H)


# speedup vs baseline: 1.2540x; 1.2540x over previous
import functools

import jax
import jax.numpy as jnp
from jax import lax
from jax.experimental import pallas as pl
from jax.experimental.pallas import tpu as pltpu
from jax.experimental.pallas import tpu_sc as plsc

B, L = 4096, 200
V, H = 1000000, 64
TOTAL = B * L
N_AGE, N_POS, N_PT = 100, 200, 3
N_AGE_PAD = 104
N_TAB = N_PT * N_POS * N_AGE


def _t2v_body(wa_ref, ba_ref, wp_ref, bp_ref, aa_ref, ab_ref,
              ta_ref, tp_ref):
    aw = aa_ref[0, 0]
    a = lax.broadcasted_iota(jnp.int32, (N_AGE_PAD, H), 0).astype(jnp.float32)
    cola = lax.broadcasted_iota(jnp.int32, (N_AGE_PAD, H), 1)
    arg_a = a * wa_ref[...] + ba_ref[...]
    ta_ref[...] = aw * jnp.where(cola == H - 1, arg_a, jnp.cos(arg_a))
    pw = ab_ref[0, 0]
    p = lax.broadcasted_iota(jnp.int32, (N_POS, H), 0).astype(jnp.float32)
    colp = lax.broadcasted_iota(jnp.int32, (N_POS, H), 1)
    arg_p = p * wp_ref[...] + bp_ref[...]
    tp_ref[...] = pw * jnp.where(colp == H - 1, arg_p, jnp.sin(arg_p))


def _build_t2v(wa, ba, wp, bp, aa, ab):
    w_spec = pl.BlockSpec((1, H), lambda: (0, 0))
    s_spec = pl.BlockSpec((1, 1), lambda: (0, 0), memory_space=pltpu.SMEM)
    return pl.pallas_call(
        _t2v_body,
        in_specs=[w_spec, w_spec, w_spec, w_spec, s_spec, s_spec],
        out_specs=[pl.BlockSpec((N_AGE_PAD, H), lambda: (0, 0)),
                   pl.BlockSpec((N_POS, H), lambda: (0, 0))],
        out_shape=[jax.ShapeDtypeStruct((N_AGE_PAD, H), jnp.float32),
                   jax.ShapeDtypeStruct((N_POS, H), jnp.float32)],
    )(wa, ba, wp, bp, aa, ab)


RT = N_POS * N_AGE
IDX_R, IDX_C = 6400, 128
IDX_GRID = 50
IDX_BLK = IDX_R // IDX_GRID


def _fuse_body(ta_ref, tp_ref, ptab_ref, dtab_ref):
    p = pl.program_id(0)
    ta = jnp.broadcast_to(ta_ref[:N_AGE, :][None], (N_POS, N_AGE, H))
    tp = jnp.broadcast_to(tp_ref[...][:, None, :], (N_POS, N_AGE, H))
    pt_rows = jnp.where(p == 0, ptab_ref[0:1, :],
                        jnp.where(p == 1, ptab_ref[1:2, :], ptab_ref[2:3, :]))
    dtab_ref[...] = (ta + tp).reshape(RT, H) + pt_rows


def _build_dense(ta, tp, ptab):
    return pl.pallas_call(
        _fuse_body,
        grid=(N_PT,),
        in_specs=[pl.BlockSpec((N_AGE_PAD, H), lambda i: (0, 0)),
                  pl.BlockSpec((N_POS, H), lambda i: (0, 0)),
                  pl.BlockSpec((8, H), lambda i: (0, 0))],
        out_specs=pl.BlockSpec((RT, H), lambda i: (i, 0)),
        out_shape=jax.ShapeDtypeStruct((N_TAB, H), jnp.float32),
    )(ta, tp, ptab)


def _fuseidx_body(age_ref, pos_ref, pt_ref, fused_ref):
    fused_ref[...] = (pt_ref[...] * (N_POS * N_AGE) + pos_ref[...] * N_AGE
                      + age_ref[...])


def _build_fused_idx(age2, pos2, pt2):
    idx_spec = pl.BlockSpec((IDX_BLK, IDX_C), lambda i: (i, 0))
    return pl.pallas_call(
        _fuseidx_body,
        grid=(IDX_GRID,),
        in_specs=[idx_spec, idx_spec, idx_spec],
        out_specs=idx_spec,
        out_shape=jax.ShapeDtypeStruct((IDX_R, IDX_C), jnp.int32),
    )(age2, pos2, pt2)


NC, NS = 2, 16
NW = NC * NS
HG, BG = H // 8, B // 128
UNITS = L * BG
U_PER_W = UNITS // NW
CU = 2
CBB = CU * 128
NCHUNK = U_PER_W // CU

_sc_mesh = plsc.VectorSubcoreMesh(core_axis_name="c", subcore_axis_name="s")


@functools.partial(
    pl.kernel,
    out_type=jax.ShapeDtypeStruct((L, HG, BG, 8, 128), jnp.float32),
    mesh=_sc_mesh,
    scratch_types=[
        pltpu.VMEM((3, CBB), jnp.int32),
        pltpu.VMEM((3, CBB), jnp.int32),
        pltpu.VMEM((3, CBB, H), jnp.float32),
        pltpu.VMEM((3, HG, CU, 8, 128), jnp.float32),
    ] + [pltpu.SemaphoreType.DMA] * 12,
    compiler_params=pltpu.CompilerParams(use_tc_tiling_on_sc=False,
                                         needs_layout_passes=False),
)
def _sc_gather(tok_hbm, fus_hbm, table_hbm, dtab_hbm, out_hbm,
               tok_v, fus_v, rows_v, t_v, *sems):
    wid = lax.axis_index("s") * NC + lax.axis_index("c")
    semI = sems[0:3]
    semA = sems[3:6]
    semB = sems[6:9]
    semC = sems[9:12]

    def u0_of(ci):
        return wid * U_PER_W + ci * CU

    def idx_copies(ci, s):
        e0 = pl.multiple_of(u0_of(ci) * 128, 8)
        return (pltpu.make_async_copy(tok_hbm.at[pl.ds(e0, CBB)], tok_v.at[s],
                                      semI[s]),
                pltpu.make_async_copy(fus_hbm.at[pl.ds(e0, CBB)], fus_v.at[s],
                                      semI[s]))

    def a_copies(s):
        return [pltpu.make_async_copy(
            table_hbm.at[tok_v.at[s]], rows_v.at[s], semA[s])]

    def b_copies(s):
        return [pltpu.make_async_copy(
            dtab_hbm.at[fus_v.at[s]], rows_v.at[s], semB[s])]

    def c_copy(ci, s):
        u0 = u0_of(ci)
        l = u0 // BG
        bg = u0 % BG
        return pltpu.make_async_copy(t_v.at[s],
                                     out_hbm.at[l, :, pl.ds(bg, CU)],
                                     semC[s])

    lane = lax.iota(jnp.int32, 16)

    def transpose(s):
        buf = rows_v.at[s]
        tb = t_v.at[s]

        def mbody(m, carry):
            h0 = m * 16

            def kbody(k, c2):
                for u in range(2):
                    bv = lane + (2 * k + u) * 16
                    bv7 = bv >> 7
                    bv127 = bv & 127
                    for d in range(16):
                        hv = h0 + ((lane + d) & 15)
                        v = plsc.load_gather(buf, [bv, hv])
                        plsc.store_scatter(tb, [hv >> 3, bv7, hv & 7, bv127],
                                           v)
                return c2

            return lax.fori_loop(0, CBB // 32, kbody, carry)

        lax.fori_loop(0, H // 16, mbody, 0)

    def fire(cps, add=False):
        for cp in (cps if isinstance(cps, (list, tuple)) else [cps]):
            cp.start(add=add)

    def wait(cps):
        for cp in (cps if isinstance(cps, (list, tuple)) else [cps]):
            cp.wait()

    def sub(ci, s, cwait=True, prefetch=True):
        sp1, sp2 = (s + 2) % 3, (s + 1) % 3
        wait(idx_copies(ci, s))
        fire(a_copies(s))
        wait(a_copies(sp1))
        fire(b_copies(sp1), add=True)
        if cwait:
            wait(c_copy(ci - 5, sp2))
        wait(b_copies(sp2))
        transpose(sp2)
        fire(c_copy(ci - 2, sp2))
        if prefetch:
            fire(idx_copies(ci + 1, sp2))

    fire(idx_copies(0, 0))
    wait(idx_copies(0, 0))
    fire(a_copies(0))
    fire(idx_copies(1, 1))
    wait(idx_copies(1, 1))
    fire(a_copies(1))
    wait(a_copies(0))
    fire(b_copies(0), add=True)
    fire(idx_copies(2, 2))
    wait(idx_copies(2, 2))
    fire(a_copies(2))
    wait(a_copies(1))
    fire(b_copies(1), add=True)
    wait(b_copies(0))
    transpose(0)
    fire(c_copy(0, 0))
    fire(idx_copies(3, 0))
    sub(3, 0, cwait=False)
    sub(4, 1, cwait=False)
    sub(5, 2)

    def body(t, carry):
        sub(3 * t + 6, 0)
        sub(3 * t + 7, 1)
        sub(3 * t + 8, 2)
        return carry

    lax.fori_loop(0, (NCHUNK - 7) // 3, body, 0)

    last = NCHUNK - 1
    sub(last, 0, prefetch=False)
    wait(a_copies(0))
    fire(b_copies(0), add=True)
    wait(b_copies(2))
    wait(c_copy(last - 4, 2))
    transpose(2)
    fire(c_copy(last - 1, 2))
    wait(b_copies(0))
    wait(c_copy(last - 3, 0))
    transpose(0)
    fire(c_copy(last, 0))
    wait(c_copy(last - 2, 1))
    wait(c_copy(last - 1, 2))
    wait(c_copy(last, 0))


def kernel(tokens, position, age, partner_type, token_table, partner_table,
           age_w, age_b, age_w0, age_b0, abs_w, abs_b, abs_w0, abs_b0,
           res_age_alpha, res_abs_alpha):
    f32 = jnp.float32
    age2 = age.astype(jnp.int32).T.reshape(IDX_R, IDX_C)
    pos2 = position.astype(jnp.int32).T.reshape(IDX_R, IDX_C)
    pt2 = partner_type.astype(jnp.int32).T.reshape(IDX_R, IDX_C)
    tok2 = tokens.astype(jnp.int32).T.reshape(-1)
    wa = jnp.concatenate([age_w, age_w0], axis=1).astype(f32)
    ba = jnp.concatenate([age_b, age_b0], axis=1).astype(f32)
    wp = jnp.concatenate([abs_w, abs_w0], axis=1).astype(f32)
    bp = jnp.concatenate([abs_b, abs_b0], axis=1).astype(f32)
    ptab = jnp.pad(partner_table.astype(f32), ((0, 8 - 3), (0, 0)))
    aa = res_age_alpha.astype(f32).reshape(1, 1)
    ab = res_abs_alpha.astype(f32).reshape(1, 1)
    ta, tp = _build_t2v(wa, ba, wp, bp, aa, ab)
    dtab = _build_dense(ta, tp, ptab)
    fused = _build_fused_idx(age2, pos2, pt2).reshape(-1)
    out5 = _sc_gather(tok2, fused, token_table.astype(f32), dtab)
    return out5.transpose(2, 4, 0, 1, 3).reshape(B, L, H)
